# Initial kernel scaffold; baseline (speedup 1.0000x reference)
import jax
import jax.numpy as jnp
from jax.experimental import pallas as pl


def kernel(t, const_embed):
    raise NotImplementedError("write your pallas kernel here")


# speedup vs baseline: 3.1626x; 3.1626x over previous
import functools
import math

import jax
import jax.numpy as jnp
from jax import lax
from jax.experimental import pallas as pl
from jax.experimental.pallas import tpu as pltpu
from jax.experimental.pallas import tpu_sc as plsc

_NC, _NS, _L = 2, 16, 16
_NW = _NC * _NS
_B, _T, _DIM = 4096, 200, 64
_ROWS = _B * _T
_RPW = _ROWS // _NW
_CHUNK = 256
_NIDX = 128
_NCHUNK = _RPW // _CHUNK
_OUTD = _DIM + 22

_CC = (-1.0 / 3628800, 1.0 / 40320, -1.0 / 720, 1.0 / 24, -0.5, 1.0)
_SC = (-1.0 / 39916800, 1.0 / 362880, -1.0 / 5040, 1.0 / 120, -1.0 / 6, 1.0)


def _horner(coefs, z):
    acc = jnp.full((_L,), coefs[0], jnp.float32)
    for c in coefs[1:]:
        acc = acc * z + c
    return acc


def _body(t_hbm, tab_hbm, out_hbm, t_v, idx_v, out_v, gsem):
    wid = lax.axis_index("s") * _NC + lax.axis_index("c")
    wbase = wid * _RPW

    def chunk(ci, carry):
        base = wbase + ci * _CHUNK
        pltpu.sync_copy(t_hbm.at[pl.ds(base, _CHUNK)], t_v)
        for g in range(_CHUNK // _L):
            tf = t_v[pl.ds(g * _L, _L)] * 2048.0
            f = tf + 0.5
            i = f.astype(jnp.int32)
            tie = (f == i.astype(jnp.float32)) & ((i & 1) == 1)
            idx_v[g // (_NIDX // _L), pl.ds((g % (_NIDX // _L)) * _L, _L)] = (
                jnp.where(tie, i - 1, i))
        copies = []
        for j in range(_CHUNK // _NIDX):
            copies.append(pltpu.make_async_copy(
                tab_hbm.at[idx_v.at[j]],
                out_v.at[pl.ds(j * _NIDX, _NIDX)],
                gsem))
        for c in copies:
            c.start()
        for c in copies:
            c.wait()
        for g in range(_CHUNK // _L):
            tf = t_v[pl.ds(g * _L, _L)] * 2048.0
            a = tf * (math.pi / 2048.0)
            w = a - (math.pi / 2.0)
            z = w * w
            s = _horner(_CC, z)
            c = -(w * _horner(_SC, z))
            rows = lax.iota(jnp.int32, _L) + (g * _L)
            for j in range(11):
                plsc.store_scatter(out_v, [rows, jnp.full((_L,), 64 + j, jnp.int32)], s)
                plsc.store_scatter(out_v, [rows, jnp.full((_L,), 75 + j, jnp.int32)], c)
                sc = s * c
                ss = s * s
                s = sc + sc
                c = 1.0 - (ss + ss)
        pltpu.sync_copy(out_v, out_hbm.at[pl.ds(base, _CHUNK)])
        return carry

    lax.fori_loop(0, _NCHUNK, chunk, 0)


@functools.partial(jax.jit, static_argnames=())
def kernel(t, const_embed):
    tflat = t.reshape(_ROWS)
    tab128 = jnp.pad(const_embed, ((0, 0), (0, 128 - _DIM)))
    run = pl.kernel(
        _body,
        out_type=jax.ShapeDtypeStruct((_ROWS, 128), jnp.float32),
        mesh=plsc.VectorSubcoreMesh(core_axis_name="c", subcore_axis_name="s"),
        scratch_types=[
            pltpu.VMEM((_CHUNK,), jnp.float32),
            pltpu.VMEM((_CHUNK // _NIDX, _NIDX), jnp.int32),
            pltpu.VMEM((_CHUNK, 128), jnp.float32),
            pltpu.SemaphoreType.DMA,
        ],
        compiler_params=pltpu.CompilerParams(needs_layout_passes=False),
    )
    out = run(tflat, tab128)
    return out[:, :_OUTD].reshape(_B, _T, _OUTD)


# speedup vs baseline: 4.3418x; 1.3729x over previous
import functools
import math

import jax
import jax.numpy as jnp
from jax import lax
from jax.experimental import pallas as pl
from jax.experimental.pallas import tpu as pltpu
from jax.experimental.pallas import tpu_sc as plsc

_NC, _NS, _L = 2, 16, 16
_NW = _NC * _NS
_B, _T, _DIM = 4096, 200, 64
_ROWS = _B * _T
_RPW = _ROWS // _NW
_CHUNK = 256
_NIDX = 128
_NCHUNK = _RPW // _CHUNK
_OUTD = _DIM + 22

_CC = (-1.0 / 3628800, 1.0 / 40320, -1.0 / 720, 1.0 / 24, -0.5, 1.0)
_SC = (-1.0 / 39916800, 1.0 / 362880, -1.0 / 5040, 1.0 / 120, -1.0 / 6, 1.0)


def _horner(coefs, z):
    acc = jnp.full((_L,), coefs[0], jnp.float32)
    for c in coefs[1:]:
        acc = acc * z + c
    return acc


def _body(t_hbm, tab_hbm, out_hbm, t_all, idx0, idx1, out0, out1,
          gsem0, gsem1, osem0, osem1):
    wid = lax.axis_index("s") * _NC + lax.axis_index("c")
    wbase = wid * _RPW

    pltpu.sync_copy(t_hbm.at[pl.ds(wbase, _RPW)], t_all)

    def gathers(idx_b, out_b, gsem):
        return [pltpu.make_async_copy(
            tab_hbm.at[idx_b.at[pl.ds(j * _NIDX, _NIDX)]],
            out_b.at[pl.ds(j * _NIDX, _NIDX)],
            gsem) for j in range(_CHUNK // _NIDX)]

    def stage_a(ci, idx_b, out_b, gsem):
        def idx_group(g, carry):
            tf = t_all[pl.ds(ci * _CHUNK + g * _L, _L)] * 2048.0
            f = tf + 0.5
            i = f.astype(jnp.int32)
            tie = (f == i.astype(jnp.float32)) & ((i & 1) == 1)
            idx_b[pl.ds(g * _L, _L)] = jnp.where(tie, i - 1, i)
            return carry
        lax.fori_loop(0, _CHUNK // _L, idx_group, 0)
        for cp in gathers(idx_b, out_b, gsem):
            cp.start()

    def stage_b(ci, idx_b, out_b, gsem, osem):
        for cp in gathers(idx_b, out_b, gsem):
            cp.wait()
        def four_group(g, carry):
            tf = t_all[pl.ds(ci * _CHUNK + g * _L, _L)] * 2048.0
            a = tf * (math.pi / 2048.0)
            w = a - (math.pi / 2.0)
            z = w * w
            s = _horner(_CC, z)
            c = -(w * _horner(_SC, z))
            rows = lax.iota(jnp.int32, _L) + g * _L
            for j in range(11):
                plsc.store_scatter(
                    out_b, [rows, jnp.full((_L,), 64 + j, jnp.int32)], s)
                plsc.store_scatter(
                    out_b, [rows, jnp.full((_L,), 75 + j, jnp.int32)], c)
                sc = s * c
                ss = s * s
                s = sc + sc
                c = 1.0 - (ss + ss)
            return carry
        lax.fori_loop(0, _CHUNK // _L, four_group, 0)
        pltpu.make_async_copy(
            out_b, out_hbm.at[pl.ds(wbase + ci * _CHUNK, _CHUNK)], osem
        ).start()

    def wait_out(out_b, osem):
        pltpu.make_async_copy(
            out_b, out_hbm.at[pl.ds(wbase, _CHUNK)], osem).wait()

    stage_a(0, idx0, out0, gsem0)
    stage_a(1, idx1, out1, gsem1)
    stage_b(0, idx0, out0, gsem0, osem0)

    def steady(k, carry):
        c = 2 * k
        wait_out(out0, osem0)
        stage_a(c + 2, idx0, out0, gsem0)
        stage_b(c + 1, idx1, out1, gsem1, osem1)
        wait_out(out1, osem1)
        stage_a(c + 3, idx1, out1, gsem1)
        stage_b(c + 2, idx0, out0, gsem0, osem0)
        return carry

    lax.fori_loop(0, (_NCHUNK - 2) // 2, steady, 0)
    stage_b(_NCHUNK - 1, idx1, out1, gsem1, osem1)
    wait_out(out0, osem0)
    wait_out(out1, osem1)


@functools.partial(jax.jit, static_argnames=())
def kernel(t, const_embed):
    tflat = t.reshape(_ROWS)
    tab128 = jnp.pad(const_embed, ((0, 0), (0, 128 - _DIM)))
    run = pl.kernel(
        _body,
        out_type=jax.ShapeDtypeStruct((_ROWS, 128), jnp.float32),
        mesh=plsc.VectorSubcoreMesh(core_axis_name="c", subcore_axis_name="s"),
        scratch_types=[
            pltpu.VMEM((_RPW,), jnp.float32),
            pltpu.VMEM((_CHUNK,), jnp.int32),
            pltpu.VMEM((_CHUNK,), jnp.int32),
            pltpu.VMEM((_CHUNK, 128), jnp.float32),
            pltpu.VMEM((_CHUNK, 128), jnp.float32),
            pltpu.SemaphoreType.DMA,
            pltpu.SemaphoreType.DMA,
            pltpu.SemaphoreType.DMA,
            pltpu.SemaphoreType.DMA,
        ],
        compiler_params=pltpu.CompilerParams(needs_layout_passes=False),
    )
    out = run(tflat, tab128)
    return out[:, :_OUTD].reshape(_B, _T, _OUTD)
